# Initial kernel scaffold; baseline (speedup 1.0000x reference)
import jax
import jax.numpy as jnp
from jax.experimental import pallas as pl


def kernel(idx, edge_index, Gx, Gy, Gs, emb, W_ro, b_ro):
    raise NotImplementedError("write your pallas kernel here")


# speedup vs baseline: 8.6657x; 8.6657x over previous
import functools

import jax
import jax.numpy as jnp
from jax import lax
from jax.experimental import pallas as pl
from jax.experimental.pallas import tpu as pltpu
from jax.experimental.pallas import tpu_sc as plsc

NN = 10000
NP = 10240
ST = 640
NT = 16
CK = 128
NC = 79
EPT = NC * CK
NE = 160000
TT = 8
NL = 2
B = 4
VOCAB = 1000
NSTEP = TT * NL


def _sc_recurrence(src_t, dst_t, gx_t, gy_t, gs_t, emb2, xidx):
  mesh = plsc.VectorSubcoreMesh(
      core_axis_name="c", subcore_axis_name="s", num_cores=1)

  @functools.partial(
      pl.kernel,
      out_type=(
          jax.ShapeDtypeStruct((TT, B, NP), jnp.float32),
          jax.ShapeDtypeStruct((NT, EPT), jnp.float32),
      ),
      mesh=mesh,
      compiler_params=pltpu.CompilerParams(needs_layout_passes=False),
      scratch_types=[
          pltpu.HBM((B, NP), jnp.float32),
          pltpu.HBM((B, NP), jnp.float32),
          pltpu.HBM((B, NP), jnp.float32),
          pltpu.HBM((NT, B, NP), jnp.float32),
          pltpu.VMEM((EPT,), jnp.int32),
          pltpu.VMEM((EPT,), jnp.int32),
          pltpu.VMEM((EPT,), jnp.float32),
          pltpu.VMEM((EPT,), jnp.float32),
          pltpu.VMEM((EPT,), jnp.float32),
          pltpu.VMEM((EPT,), jnp.float32),
          pltpu.VMEM((EPT,), jnp.float32),
          pltpu.VMEM((TT, 8), jnp.int32),
          pltpu.VMEM((8, ST), jnp.float32),
          pltpu.VMEM((NP,), jnp.float32),
          pltpu.VMEM((NP,), jnp.float32),
          pltpu.VMEM((NP,), jnp.float32),
          pltpu.VMEM((NT, ST), jnp.float32),
          pltpu.VMEM((ST,), jnp.float32),
          pltpu.SemaphoreType.DMA,
      ],
  )
  def k(src_h, dst_h, gx_h, gy_h, gs_h, emb_h, xidx_h, xout_h, sig_h,
        xarr, yarr, aarr, part,
        src_v, dst_v, sig_v, gx_v, gy_v, gs_v, hsum, xidx_v, xtbuf,
        cola, colb, acc_col, redbuf, stbuf, sem):
    w = lax.axis_index("s")
    zeros16 = jnp.zeros((16,), jnp.float32)

    pltpu.sync_copy(src_h.at[w], src_v)
    pltpu.sync_copy(dst_h.at[w], dst_v)
    pltpu.sync_copy(gx_h.at[w], gx_v)
    pltpu.sync_copy(gy_h.at[w], gy_v)
    pltpu.sync_copy(gs_h.at[w], gs_v)
    pltpu.sync_copy(xidx_h.at[w], xidx_v)

    def sz_body(v, _):
      sig_v[pl.ds(v * 16, 16)] = zeros16
      hsum[pl.ds(v * 16, 16)] = zeros16
      return 0
    lax.fori_loop(0, EPT // 16, sz_body, 0)

    def zero_acc():
      def zb(i, _):
        acc_col[pl.ds(i * 16, 16)] = zeros16
        return 0
      lax.fori_loop(0, NP // 16, zb, 0)

    def reduce_col(b, dst_arr, do_relu, t, do_export):
      cps = [pltpu.async_copy(part.at[k2, b, pl.ds(w * ST, ST)],
                              redbuf.at[k2], sem) for k2 in range(NT)]
      for cp in cps:
        cp.wait()

      def rb(i, _):
        d = pl.ds(i * 16, 16)
        a = redbuf[0, d]
        for k2 in range(1, NT):
          a = a + redbuf[k2, d]
        if do_relu:
          a = jnp.maximum(a, 0.0)
        stbuf[d] = a
        return 0
      lax.fori_loop(0, ST // 16, rb, 0)
      pltpu.sync_copy(stbuf, dst_arr.at[b, pl.ds(w * ST, ST)])
      if do_export:
        @pl.when(do_export[0])
        def _():
          pltpu.sync_copy(stbuf, xout_h.at[t, b, pl.ds(w * ST, ST)])

    def step_body(s, _):
      t = lax.shift_right_logical(s, 1)
      layer = lax.bitwise_and(s, 1)

      @pl.when(layer == 0)
      def _():
        pltpu.async_copy(emb_h.at[xidx_v.at[t]], xtbuf, sem).wait()
        for b in range(B):
          def xc(i, _):
            d = pl.ds(i * 16, 16)
            stbuf[d] = xtbuf[b, d]
            return 0
          lax.fori_loop(0, ST // 16, xc, 0)
          pltpu.sync_copy(stbuf, xarr.at[b, pl.ds(w * ST, ST)])

          @pl.when(s == 0)
          def _():
            pltpu.sync_copy(stbuf, yarr.at[b, pl.ds(w * ST, ST)])
      plsc.subcore_barrier()

      for b in range(B):
        cpa = pltpu.async_copy(xarr.at[b], cola, sem)
        cpb = pltpu.async_copy(yarr.at[b], colb, sem)
        cpa.wait()
        cpb.wait()
        zero_acc()

        def l1c(c, _):
          for u in range(CK // 16):
            d = pl.ds(c * CK + u * 16, 16)
            s16 = src_v[d]
            d16 = dst_v[d]
            xs = plsc.load_gather(cola, [s16])
            plsc.addupdate_scatter(acc_col, [d16], xs * sig_v[d])
            ys = plsc.load_gather(colb, [s16])
            xd = plsc.load_gather(cola, [d16])
            hsum[d] = hsum[d] + ys * xd
          return 0
        lax.fori_loop(0, NC, l1c, 0)
        pltpu.sync_copy(acc_col, part.at[w, b])

      def su(v, _):
        d = pl.ds(v * 16, 16)
        sig_v[d] = (sig_v[d] + hsum[d] * 0.25 * gs_v[d]) * 0.99
        hsum[d] = zeros16
        return 0
      lax.fori_loop(0, EPT // 16, su, 0)
      plsc.subcore_barrier()

      for b in range(B):
        reduce_col(b, aarr, False, t, None)
      plsc.subcore_barrier()

      for b in range(B):
        pltpu.async_copy(aarr.at[b], cola, sem).wait()
        zero_acc()

        def l2c(c, _):
          for u in range(CK // 16):
            d = pl.ds(c * CK + u * 16, 16)
            s16 = src_v[d]
            d16 = dst_v[d]
            av = jnp.maximum(plsc.load_gather(cola, [s16]), 0.0)
            plsc.addupdate_scatter(acc_col, [d16], av * gy_v[d])
          return 0
        lax.fori_loop(0, NC, l2c, 0)
        pltpu.sync_copy(acc_col, part.at[w, b])
      plsc.subcore_barrier()

      for b in range(B):
        reduce_col(b, yarr, False, t, None)
      plsc.subcore_barrier()

      for b in range(B):
        pltpu.async_copy(yarr.at[b], cola, sem).wait()
        zero_acc()

        def l3c(c, _):
          for u in range(CK // 16):
            d = pl.ds(c * CK + u * 16, 16)
            s16 = src_v[d]
            d16 = dst_v[d]
            yv = plsc.load_gather(cola, [s16])
            plsc.addupdate_scatter(acc_col, [d16], yv * gx_v[d])
          return 0
        lax.fori_loop(0, NC, l3c, 0)
        pltpu.sync_copy(acc_col, part.at[w, b])
      plsc.subcore_barrier()

      for b in range(B):
        reduce_col(b, xarr, True, t, (layer == 1,))
      plsc.subcore_barrier()
      return 0

    lax.fori_loop(0, NSTEP, step_body, 0)

    pltpu.sync_copy(sig_v, sig_h.at[w])

  return k(src_t, dst_t, gx_t, gy_t, gs_t, emb2, xidx)


def _tc_readout(xf, w_ro, b_ro):
  blk = 2048
  nblk = NP // blk

  def body(x_ref, w_ref, b_ref, o_ref):
    @pl.when(pl.program_id(0) == 0)
    def _():
      o_ref[...] = jnp.broadcast_to(b_ref[...][None, None, :],
                                    (B, TT, VOCAB))
    xs = x_ref[...]
    ws = w_ref[...]
    for t in range(TT):
      o_ref[:, t, :] += lax.dot_general(xs[t], ws, (((1,), (0,)), ((), ())),
                                        preferred_element_type=jnp.float32)

  return pl.pallas_call(
      body,
      grid=(nblk,),
      in_specs=[
          pl.BlockSpec((TT, B, blk), lambda i: (0, 0, i)),
          pl.BlockSpec((blk, VOCAB), lambda i: (i, 0)),
          pl.BlockSpec((VOCAB,), lambda i: (0,)),
      ],
      out_specs=pl.BlockSpec((B, TT, VOCAB), lambda i: (0, 0, 0)),
      out_shape=jax.ShapeDtypeStruct((B, TT, VOCAB), jnp.float32),
  )(xf, w_ro, b_ro)


def kernel(idx, edge_index, Gx, Gy, Gs, emb, W_ro, b_ro):
  idx = idx.astype(jnp.int32)
  src = edge_index[0].astype(jnp.int32)
  dst = edge_index[1].astype(jnp.int32)
  pad = NT * EPT - NE

  def pad_e(a):
    return jnp.concatenate([a, jnp.zeros((pad,), a.dtype)]).reshape(NT, EPT)

  src_t = pad_e(src)
  dst_t = pad_e(dst)
  gx_t = pad_e(Gx)
  gy_t = pad_e(Gy)
  gs_t = pad_e(Gs)

  emb2 = jnp.pad(emb, ((0, 0), (0, NP - NN))).reshape(VOCAB * NT, ST)
  xidx4 = (idx.T[None, :, :] * NT
           + jnp.arange(NT, dtype=jnp.int32)[:, None, None])
  xidx = jnp.concatenate([xidx4, xidx4], axis=2)

  xf, sig_out = _sc_recurrence(src_t, dst_t, gx_t, gy_t, gs_t, emb2, xidx)
  w_pad = jnp.pad(W_ro, ((0, NP - NN), (0, 0)))
  logits = _tc_readout(xf, w_pad, b_ro)
  sigma = sig_out.reshape(-1)[:NE]
  return (logits, jax.lax.stop_gradient(sigma))


# speedup vs baseline: 8.6770x; 1.0013x over previous
import functools

import jax
import jax.numpy as jnp
from jax import lax
from jax.experimental import pallas as pl
from jax.experimental.pallas import tpu as pltpu
from jax.experimental.pallas import tpu_sc as plsc

NN = 10000
NP = 10240
ST = 640
NT = 16
CK = 128
NC = 79
EPT = NC * CK
NE = 160000
TT = 8
NL = 2
B = 4
VOCAB = 1000
NSTEP = TT * NL


def _sc_recurrence(src_t, dst_t, gx_t, gy_t, gs_t, emb2, xidx):
  mesh = plsc.VectorSubcoreMesh(
      core_axis_name="c", subcore_axis_name="s", num_cores=1)

  @functools.partial(
      pl.kernel,
      out_type=(
          jax.ShapeDtypeStruct((TT, B, NP), jnp.float32),
          jax.ShapeDtypeStruct((NT, EPT), jnp.float32),
      ),
      mesh=mesh,
      compiler_params=pltpu.CompilerParams(needs_layout_passes=False),
      scratch_types=[
          pltpu.HBM((B, NP), jnp.float32),
          pltpu.HBM((B, NP), jnp.float32),
          pltpu.HBM((B, NP), jnp.float32),
          pltpu.HBM((NT, B, NP), jnp.float32),
          pltpu.VMEM((EPT,), jnp.int32),
          pltpu.VMEM((EPT,), jnp.int32),
          pltpu.VMEM((EPT,), jnp.float32),
          pltpu.VMEM((EPT,), jnp.float32),
          pltpu.VMEM((EPT,), jnp.float32),
          pltpu.VMEM((EPT,), jnp.float32),
          pltpu.VMEM((EPT,), jnp.float32),
          pltpu.VMEM((TT, 8), jnp.int32),
          pltpu.VMEM((8, ST), jnp.float32),
          pltpu.VMEM((NP,), jnp.float32),
          pltpu.VMEM((NP,), jnp.float32),
          pltpu.VMEM((NP,), jnp.float32),
          pltpu.VMEM((NT, ST), jnp.float32),
          pltpu.VMEM((ST,), jnp.float32),
          pltpu.SemaphoreType.DMA,
      ],
  )
  def k(src_h, dst_h, gx_h, gy_h, gs_h, emb_h, xidx_h, xout_h, sig_h,
        xarr, yarr, aarr, part,
        src_v, dst_v, sig_v, gx_v, gy_v, gs_v, hsum, xidx_v, xtbuf,
        cola, colb, acc_col, redbuf, stbuf, sem):
    w = lax.axis_index("s")
    zeros16 = jnp.zeros((16,), jnp.float32)

    pltpu.sync_copy(src_h.at[w], src_v)
    pltpu.sync_copy(dst_h.at[w], dst_v)
    pltpu.sync_copy(gx_h.at[w], gx_v)
    pltpu.sync_copy(gy_h.at[w], gy_v)
    pltpu.sync_copy(gs_h.at[w], gs_v)
    pltpu.sync_copy(xidx_h.at[w], xidx_v)

    def sz_body(v, _):
      sig_v[pl.ds(v * 16, 16)] = zeros16
      hsum[pl.ds(v * 16, 16)] = zeros16
      return 0
    lax.fori_loop(0, EPT // 16, sz_body, 0)

    def zero_acc():
      def zb(i, _):
        acc_col[pl.ds(i * 16, 16)] = zeros16
        return 0
      lax.fori_loop(0, NP // 16, zb, 0)

    def reduce_col(b, dst_arr, do_relu, t, do_export):
      pltpu.async_copy(part.at[:, b, pl.ds(w * ST, ST)], redbuf, sem).wait()

      def rb(i, _):
        d = pl.ds(i * 16, 16)
        a = redbuf[0, d]
        for k2 in range(1, NT):
          a = a + redbuf[k2, d]
        if do_relu:
          a = jnp.maximum(a, 0.0)
        stbuf[d] = a
        return 0
      lax.fori_loop(0, ST // 16, rb, 0)
      pltpu.sync_copy(stbuf, dst_arr.at[b, pl.ds(w * ST, ST)])
      if do_export:
        @pl.when(do_export[0])
        def _():
          pltpu.sync_copy(stbuf, xout_h.at[t, b, pl.ds(w * ST, ST)])

    def step_body(s, _):
      t = lax.shift_right_logical(s, 1)
      layer = lax.bitwise_and(s, 1)

      @pl.when(layer == 0)
      def _():
        pltpu.async_copy(emb_h.at[xidx_v.at[t]], xtbuf, sem).wait()
        for b in range(B):
          def xc(i, _):
            d = pl.ds(i * 16, 16)
            stbuf[d] = xtbuf[b, d]
            return 0
          lax.fori_loop(0, ST // 16, xc, 0)
          pltpu.sync_copy(stbuf, xarr.at[b, pl.ds(w * ST, ST)])

          @pl.when(s == 0)
          def _():
            pltpu.sync_copy(stbuf, yarr.at[b, pl.ds(w * ST, ST)])
      plsc.subcore_barrier()

      for b in range(B):
        cpa = pltpu.async_copy(xarr.at[b], cola, sem)
        cpb = pltpu.async_copy(yarr.at[b], colb, sem)
        cpa.wait()
        cpb.wait()
        zero_acc()

        def l1c(c, _):
          for u in range(CK // 16):
            d = pl.ds(c * CK + u * 16, 16)
            s16 = src_v[d]
            d16 = dst_v[d]
            xs = plsc.load_gather(cola, [s16])
            plsc.addupdate_scatter(acc_col, [d16], xs * sig_v[d])
            ys = plsc.load_gather(colb, [s16])
            xd = plsc.load_gather(cola, [d16])
            hsum[d] = hsum[d] + ys * xd
          return 0
        lax.fori_loop(0, NC, l1c, 0)
        pltpu.sync_copy(acc_col, part.at[w, b])

      def su(v, _):
        d = pl.ds(v * 16, 16)
        sig_v[d] = (sig_v[d] + hsum[d] * 0.25 * gs_v[d]) * 0.99
        hsum[d] = zeros16
        return 0
      lax.fori_loop(0, EPT // 16, su, 0)
      plsc.subcore_barrier()

      for b in range(B):
        reduce_col(b, aarr, False, t, None)
      plsc.subcore_barrier()

      for b in range(B):
        pltpu.async_copy(aarr.at[b], cola, sem).wait()
        zero_acc()

        def l2c(c, _):
          for u in range(CK // 16):
            d = pl.ds(c * CK + u * 16, 16)
            s16 = src_v[d]
            d16 = dst_v[d]
            av = jnp.maximum(plsc.load_gather(cola, [s16]), 0.0)
            plsc.addupdate_scatter(acc_col, [d16], av * gy_v[d])
          return 0
        lax.fori_loop(0, NC, l2c, 0)
        pltpu.sync_copy(acc_col, part.at[w, b])
      plsc.subcore_barrier()

      for b in range(B):
        reduce_col(b, yarr, False, t, None)
      plsc.subcore_barrier()

      for b in range(B):
        pltpu.async_copy(yarr.at[b], cola, sem).wait()
        zero_acc()

        def l3c(c, _):
          for u in range(CK // 16):
            d = pl.ds(c * CK + u * 16, 16)
            s16 = src_v[d]
            d16 = dst_v[d]
            yv = plsc.load_gather(cola, [s16])
            plsc.addupdate_scatter(acc_col, [d16], yv * gx_v[d])
          return 0
        lax.fori_loop(0, NC, l3c, 0)
        pltpu.sync_copy(acc_col, part.at[w, b])
      plsc.subcore_barrier()

      for b in range(B):
        reduce_col(b, xarr, True, t, (layer == 1,))
      plsc.subcore_barrier()
      return 0

    lax.fori_loop(0, NSTEP, step_body, 0)

    pltpu.sync_copy(sig_v, sig_h.at[w])

  return k(src_t, dst_t, gx_t, gy_t, gs_t, emb2, xidx)


def _tc_readout(xf, w_ro, b_ro):
  blk = 2048
  nblk = NP // blk

  def body(x_ref, w_ref, b_ref, o_ref):
    @pl.when(pl.program_id(0) == 0)
    def _():
      o_ref[...] = jnp.broadcast_to(b_ref[...][None, None, :],
                                    (B, TT, VOCAB))
    xs = x_ref[...]
    ws = w_ref[...]
    for t in range(TT):
      o_ref[:, t, :] += lax.dot_general(xs[t], ws, (((1,), (0,)), ((), ())),
                                        preferred_element_type=jnp.float32)

  return pl.pallas_call(
      body,
      grid=(nblk,),
      in_specs=[
          pl.BlockSpec((TT, B, blk), lambda i: (0, 0, i)),
          pl.BlockSpec((blk, VOCAB), lambda i: (i, 0)),
          pl.BlockSpec((VOCAB,), lambda i: (0,)),
      ],
      out_specs=pl.BlockSpec((B, TT, VOCAB), lambda i: (0, 0, 0)),
      out_shape=jax.ShapeDtypeStruct((B, TT, VOCAB), jnp.float32),
  )(xf, w_ro, b_ro)


def kernel(idx, edge_index, Gx, Gy, Gs, emb, W_ro, b_ro):
  idx = idx.astype(jnp.int32)
  src = edge_index[0].astype(jnp.int32)
  dst = edge_index[1].astype(jnp.int32)
  pad = NT * EPT - NE

  def pad_e(a):
    return jnp.concatenate([a, jnp.zeros((pad,), a.dtype)]).reshape(NT, EPT)

  src_t = pad_e(src)
  dst_t = pad_e(dst)
  gx_t = pad_e(Gx)
  gy_t = pad_e(Gy)
  gs_t = pad_e(Gs)

  emb2 = jnp.pad(emb, ((0, 0), (0, NP - NN))).reshape(VOCAB * NT, ST)
  xidx4 = (idx.T[None, :, :] * NT
           + jnp.arange(NT, dtype=jnp.int32)[:, None, None])
  xidx = jnp.concatenate([xidx4, xidx4], axis=2)

  xf, sig_out = _sc_recurrence(src_t, dst_t, gx_t, gy_t, gs_t, emb2, xidx)
  w_pad = jnp.pad(W_ro, ((0, NP - NN), (0, 0)))
  logits = _tc_readout(xf, w_pad, b_ro)
  sigma = sig_out.reshape(-1)[:NE]
  return (logits, jax.lax.stop_gradient(sigma))
